# Initial kernel scaffold; baseline (speedup 1.0000x reference)
import functools

import jax
import jax.numpy as jnp
from jax import lax
from jax.experimental import pallas as pl
from jax.experimental.pallas import tpu as pltpu
from jax.experimental.pallas import tpu_sc as plsc

_NC = 2
_NS = 16
_NW = _NC * _NS
_L = 16

_MAX_NORM = 10.0
_EPS = 1e-7


def _rsqrt_nr(x):
    i = lax.bitcast_convert_type(x, jnp.int32)
    i = jnp.int32(0x5F3759DF) - (i >> 1)
    y = lax.bitcast_convert_type(i, jnp.float32)
    for _ in range(3):
        y = y * (1.5 - 0.5 * x * y * y)
    return y


def _clip_scale(ssv):
    ss = jnp.maximum(ssv, jnp.float32(_EPS * _EPS))
    return jnp.minimum(jnp.float32(1.0), jnp.float32(_MAX_NORM) * _rsqrt_nr(ss))


@functools.cache
def _build(B, H, D, n_in, n_out):
    assert D == 2 * _L and B % _NW == 0
    bpw = B // _NW
    och = bpw // 128
    assert och * 128 == bpw

    mesh = plsc.VectorSubcoreMesh(
        core_axis_name="c", subcore_axis_name="s",
        num_cores=_NC, num_subcores=_NS)

    def body(inp_ref, oidx_ref, win_ref, wout_ref, o1_ref, o2_ref,
             idx_v, rows_v, oidx_v, orows_v, out_v, sem_g, sem_o):
        wid = lax.axis_index("s") * _NC + lax.axis_index("c")
        base = wid * bpw

        pltpu.sync_copy(inp_ref.at[pl.ds(base, bpw)], idx_v)
        pltpu.sync_copy(oidx_ref.at[wid], oidx_v)

        for c in range(och):
            pltpu.async_copy(wout_ref.at[oidx_v.at[c]],
                             orows_v.at[pl.ds(c * 128, 128)], sem_o)

        pltpu.async_copy(win_ref.at[idx_v.at[0]], rows_v.at[0], sem_g.at[0])

        def bstep(b, _):
            p = lax.rem(b, 2)
            pltpu.make_async_copy(win_ref.at[idx_v.at[b]], rows_v.at[p],
                                  sem_g.at[p]).wait()

            @pl.when(b < bpw - 1)
            def _prefetch():
                pltpu.async_copy(win_ref.at[idx_v.at[b + 1]],
                                 rows_v.at[1 - p], sem_g.at[1 - p])

            def lstep(l, carry):
                a0, a1 = carry
                r0 = rows_v[p, l, pl.ds(0, _L)]
                r1 = rows_v[p, l, pl.ds(_L, _L)]
                ss = jnp.sum(r0 * r0 + r1 * r1)
                scale = _clip_scale(jnp.full((_L,), ss, jnp.float32))
                return (a0 + scale * r0, a1 + scale * r1)

            z = jnp.zeros((_L,), jnp.float32)
            a0, a1 = lax.fori_loop(0, H, lstep, (z, z))
            out_v[b, pl.ds(0, _L)] = a0
            out_v[b, pl.ds(_L, _L)] = a1
            return 0

        lax.fori_loop(0, bpw, bstep, 0)

        for c in range(och):
            pltpu.make_async_copy(wout_ref.at[oidx_v.at[c]],
                                  orows_v.at[pl.ds(c * 128, 128)], sem_o).wait()

        def ostep(r, _):
            r0 = orows_v[r, pl.ds(0, _L)]
            r1 = orows_v[r, pl.ds(_L, _L)]
            ss = jnp.sum(r0 * r0 + r1 * r1)
            scale = _clip_scale(jnp.full((_L,), ss, jnp.float32))
            orows_v[r, pl.ds(0, _L)] = scale * r0
            orows_v[r, pl.ds(_L, _L)] = scale * r1
            return 0

        lax.fori_loop(0, bpw, ostep, 0)

        pltpu.sync_copy(out_v, o1_ref.at[pl.ds(base, bpw)])
        pltpu.sync_copy(orows_v, o2_ref.at[pl.ds(base, bpw)])

    return pl.kernel(
        body,
        out_type=(jax.ShapeDtypeStruct((B, D), jnp.float32),
                  jax.ShapeDtypeStruct((B, D), jnp.float32)),
        mesh=mesh,
        scratch_types=[
            pltpu.VMEM((bpw, H), jnp.int32),
            pltpu.VMEM((2, H, D), jnp.float32),
            pltpu.VMEM((bpw // 128, 128), jnp.int32),
            pltpu.VMEM((bpw, D), jnp.float32),
            pltpu.VMEM((bpw, D), jnp.float32),
            pltpu.SemaphoreType.DMA((2,)),
            pltpu.SemaphoreType.DMA,
        ],
    )


def kernel(input, output, W_in, W_out):
    B, H = input.shape
    n_in, D = W_in.shape
    n_out = W_out.shape[0]
    bpw = B // _NW
    fn = _build(B, H, D, n_in, n_out)
    oidx = output.astype(jnp.int32).reshape(_NW, bpw // 128, 128)
    return fn(input.astype(jnp.int32), oidx, W_in, W_out)


# speedup vs baseline: 1.4188x; 1.4188x over previous
import functools

import jax
import jax.numpy as jnp
from jax import lax
from jax.experimental import pallas as pl
from jax.experimental.pallas import tpu as pltpu
from jax.experimental.pallas import tpu_sc as plsc

_NC = 2
_NS = 16
_NW = _NC * _NS
_L = 16

_MAX_NORM = 10.0
_EPS = 1e-7


def _lane_allreduce_sum(x):
    lanes = lax.iota(jnp.int32, 16)
    for k in (8, 4, 2, 1):
        x = x + x.at[lanes ^ k].get(mode="promise_in_bounds")
    return x


def _rsqrt_nr(x):
    i = lax.bitcast_convert_type(x, jnp.int32)
    i = jnp.int32(0x5F3759DF) - (i >> 1)
    y = lax.bitcast_convert_type(i, jnp.float32)
    for _ in range(3):
        y = y * (1.5 - 0.5 * x * y * y)
    return y


def _clip_scale(ssv):
    ss = jnp.maximum(ssv, jnp.float32(_EPS * _EPS))
    return jnp.minimum(jnp.float32(1.0), jnp.float32(_MAX_NORM) * _rsqrt_nr(ss))


@functools.cache
def _build(B, H, D, n_in, n_out):
    assert D == 2 * _L and B % _NW == 0
    bpw = B // _NW
    och = bpw // 128
    assert och * 128 == bpw

    mesh = plsc.VectorSubcoreMesh(
        core_axis_name="c", subcore_axis_name="s",
        num_cores=_NC, num_subcores=_NS)

    def body(inp_ref, oidx_ref, win_ref, wout_ref, o1_ref, o2_ref,
             idx_v, rows_v, oidx_v, orows_v, out_v, sem_g, sem_o):
        wid = lax.axis_index("s") * _NC + lax.axis_index("c")
        base = wid * bpw

        pltpu.sync_copy(inp_ref.at[pl.ds(base, bpw)], idx_v)
        pltpu.sync_copy(oidx_ref.at[wid], oidx_v)

        for c in range(och):
            pltpu.async_copy(wout_ref.at[oidx_v.at[c]],
                             orows_v.at[pl.ds(c * 128, 128)], sem_o)

        pltpu.async_copy(win_ref.at[idx_v.at[0]], rows_v.at[0], sem_g.at[0])

        def bstep(b, _):
            p = lax.rem(b, 2)
            pltpu.make_async_copy(win_ref.at[idx_v.at[b]], rows_v.at[p],
                                  sem_g.at[p]).wait()

            @pl.when(b < bpw - 1)
            def _prefetch():
                pltpu.async_copy(win_ref.at[idx_v.at[b + 1]],
                                 rows_v.at[1 - p], sem_g.at[1 - p])

            def lstep(l, carry):
                a0, a1 = carry
                r0 = rows_v[p, l, pl.ds(0, _L)]
                r1 = rows_v[p, l, pl.ds(_L, _L)]
                ss = _lane_allreduce_sum(r0 * r0 + r1 * r1)
                scale = _clip_scale(ss)
                return (a0 + scale * r0, a1 + scale * r1)

            z = jnp.zeros((_L,), jnp.float32)
            a0, a1 = lax.fori_loop(0, H, lstep, (z, z))
            out_v[b, pl.ds(0, _L)] = a0
            out_v[b, pl.ds(_L, _L)] = a1
            return 0

        lax.fori_loop(0, bpw, bstep, 0)

        for c in range(och):
            pltpu.make_async_copy(wout_ref.at[oidx_v.at[c]],
                                  orows_v.at[pl.ds(c * 128, 128)], sem_o).wait()

        def ostep(r, _):
            r0 = orows_v[r, pl.ds(0, _L)]
            r1 = orows_v[r, pl.ds(_L, _L)]
            ss = _lane_allreduce_sum(r0 * r0 + r1 * r1)
            scale = _clip_scale(ss)
            orows_v[r, pl.ds(0, _L)] = scale * r0
            orows_v[r, pl.ds(_L, _L)] = scale * r1
            return 0

        lax.fori_loop(0, bpw, ostep, 0)

        pltpu.sync_copy(out_v, o1_ref.at[pl.ds(base, bpw)])
        pltpu.sync_copy(orows_v, o2_ref.at[pl.ds(base, bpw)])

    return pl.kernel(
        body,
        out_type=(jax.ShapeDtypeStruct((B, D), jnp.float32),
                  jax.ShapeDtypeStruct((B, D), jnp.float32)),
        mesh=mesh,
        compiler_params=pltpu.CompilerParams(use_tc_tiling_on_sc=False),
        scratch_types=[
            pltpu.VMEM((bpw, H), jnp.int32),
            pltpu.VMEM((2, H, D), jnp.float32),
            pltpu.VMEM((bpw // 128, 128), jnp.int32),
            pltpu.VMEM((bpw, D), jnp.float32),
            pltpu.VMEM((bpw, D), jnp.float32),
            pltpu.SemaphoreType.DMA((2,)),
            pltpu.SemaphoreType.DMA,
        ],
    )


def kernel(input, output, W_in, W_out):
    B, H = input.shape
    n_in, D = W_in.shape
    n_out = W_out.shape[0]
    bpw = B // _NW
    fn = _build(B, H, D, n_in, n_out)
    oidx = output.astype(jnp.int32).reshape(_NW, bpw // 128, 128)
    return fn(input.astype(jnp.int32), oidx, W_in, W_out)
